# Initial kernel scaffold; baseline (speedup 1.0000x reference)
import jax
import jax.numpy as jnp
from jax.experimental import pallas as pl


def kernel(dst_nodes, root_ts, root_edge_feat, memory, memory_ts, mail, mail_ts, W_ih, b_ih, W_hh, b_hh, time_w, time_b, tl_W, tl_b, ln_g, ln_b, ep_src_W, ep_src_b, ep_dst_W, ep_dst_b, ep_out_W, ep_out_b):
    raise NotImplementedError("write your pallas kernel here")


# speedup vs baseline: 3.7651x; 3.7651x over previous
import dataclasses
import functools

import jax
import jax.numpy as jnp
from jax import lax
from jax.experimental import pallas as pl
from jax.experimental.pallas import tpu as pltpu
from jax.experimental.pallas import tpu_sc as plsc

NUM_NODES = 100000
B = 4096
DIM_EMBED = 128
DIM_EDGE = 16
DIM_TIME = 100
MAIL_DIM = DIM_EMBED + DIM_EDGE
N3 = 3 * B
N2 = 2 * B
NW = 32
GCH = N3 // (NW * 128)
SCH = N2 // (NW * 128)

_mesh = plsc.VectorSubcoreMesh(core_axis_name="c", subcore_axis_name="s",
                               num_cores=2, num_subcores=16)

f32 = jnp.float32
i32 = jnp.int32


def _wid():
    return lax.axis_index("s") * 2 + lax.axis_index("c")


@functools.partial(
    pl.kernel,
    out_type=(
        jax.ShapeDtypeStruct((N3, DIM_EMBED), f32),
        jax.ShapeDtypeStruct((N3, MAIL_DIM), f32),
        jax.ShapeDtypeStruct((N3,), f32),
        jax.ShapeDtypeStruct((N3,), f32),
    ),
    mesh=_mesh,
    scratch_types=[
        pltpu.VMEM((GCH, 128), i32),
        pltpu.VMEM((128, DIM_EMBED), f32),
        pltpu.VMEM((128, MAIL_DIM), f32),
        pltpu.VMEM((128,), f32),
        pltpu.VMEM((128,), f32),
        pltpu.SemaphoreType.DMA,
    ],
)
def _sc_gather(mem_hbm, mail_hbm, mts_hbm, lts_hbm, idx_hbm,
               pm_hbm, m_hbm, pts_hbm, mtsg_hbm,
               idx_v, rows_v, mrows_v, ts1_v, ts2_v, sem):
    w = _wid()
    pltpu.sync_copy(idx_hbm.at[w], idx_v)
    for j in range(GCH):
        base = (w * GCH + j) * 128
        pltpu.sync_copy(mem_hbm.at[idx_v.at[j]], rows_v)
        pltpu.sync_copy(rows_v, pm_hbm.at[pl.ds(base, 128), :])
        pltpu.sync_copy(mts_hbm.at[idx_v.at[j]], ts1_v)
        pltpu.sync_copy(ts1_v, pts_hbm.at[pl.ds(base, 128)])
        pltpu.sync_copy(lts_hbm.at[idx_v.at[j]], ts2_v)
        pltpu.sync_copy(ts2_v, mtsg_hbm.at[pl.ds(base, 128)])

        @pl.loop(0, 128, step=16)
        def _(l0):
            l0 = pl.multiple_of(l0, 16)
            v = idx_v[j, pl.ds(l0, 16)]
            for k in range(16):
                pltpu.async_copy(mail_hbm.at[v[k]], mrows_v.at[l0 + k], sem)

        @pl.loop(0, 128)
        def _(l):
            pltpu.make_async_copy(mail_hbm.at[0], mrows_v.at[0], sem).wait()

        pltpu.sync_copy(mrows_v, m_hbm.at[pl.ds(base, 128), :])


_GD = lax.GatherDimensionNumbers(
    offset_dims=(), collapsed_slice_dims=(0,), start_index_map=(0,))


_cp_no_layout = pltpu.CompilerParams()
if "needs_layout_passes" in pltpu.CompilerParams.__dataclass_fields__:
    _cp_no_layout = dataclasses.replace(_cp_no_layout, needs_layout_passes=False)


@functools.partial(
    pl.kernel,
    out_type=jax.ShapeDtypeStruct((N2,), i32),
    mesh=_mesh,
    compiler_params=_cp_no_layout,
    scratch_types=[
        pltpu.VMEM((N2,), i32),
        pltpu.VMEM((NUM_NODES,), i32),
        pltpu.VMEM((N2,), i32),
    ],
)
def _sc_lastpos(idx_hbm, lp_hbm, idx_v, tbl_v, lp_v):
    w = _wid()

    @pl.when(w == 0)
    def _():
        pltpu.sync_copy(idx_hbm, idx_v)
        iota = lax.iota(i32, 16)
        cidx = jnp.minimum(iota + 1, 15)
        is_last = iota == 15

        @pl.loop(0, N2, step=16)
        def _(c):
            c = pl.multiple_of(c, 16)
            k = idx_v[pl.ds(c, 16)]
            comb = (k << 13) | (iota + c)
            comb, _unused = plsc.sort_key_val(comb, comb)
            nid = comb >> 13
            npos = comb & 8191
            nxt = lax.gather(nid, cidx[:, None], _GD, slice_sizes=(1,),
                             mode=lax.GatherScatterMode.PROMISE_IN_BOUNDS)
            mask = (nid != nxt) | is_last
            plsc.store_scatter(tbl_v, [nid], npos, mask=mask)

        @pl.loop(0, N2, step=16)
        def _(c):
            c = pl.multiple_of(c, 16)
            k = idx_v[pl.ds(c, 16)]
            lp_v[pl.ds(c, 16)] = plsc.load_gather(tbl_v, [k])

        pltpu.sync_copy(lp_v, lp_hbm)


@functools.partial(
    pl.kernel,
    out_type=(),
    mesh=_mesh,
    scratch_types=[
        pltpu.VMEM((SCH, 128), i32),
        pltpu.VMEM((SCH, 128), i32),
        pltpu.VMEM((128, DIM_EMBED), f32),
        pltpu.VMEM((128,), f32),
        pltpu.VMEM((128,), f32),
        pltpu.SemaphoreType.DMA,
    ],
)
def _sc_scatter(mem_ref, mail_ref, mts_ref, lts_ref,
                nrm_hbm, mrow_hbm, mtsv_hbm, rt2_hbm, idx_hbm, lp_hbm,
                idx_v, lp_v, rows_v, ts1_v, ts2_v, sem):
    w = _wid()
    pltpu.sync_copy(idx_hbm.at[w], idx_v)
    pltpu.sync_copy(lp_hbm.at[w], lp_v)

    for j in range(SCH):
        pltpu.sync_copy(nrm_hbm.at[lp_v.at[j]], rows_v)
        pltpu.sync_copy(rows_v, mem_ref.at[idx_v.at[j]])
        pltpu.sync_copy(mtsv_hbm.at[lp_v.at[j]], ts1_v)
        pltpu.sync_copy(ts1_v, mts_ref.at[idx_v.at[j]])
        pltpu.sync_copy(rt2_hbm.at[lp_v.at[j]], ts2_v)
        pltpu.sync_copy(ts2_v, lts_ref.at[idx_v.at[j]])

        @pl.loop(0, 128, step=16)
        def _(l0):
            l0 = pl.multiple_of(l0, 16)
            vl = lp_v[j, pl.ds(l0, 16)]
            vi = idx_v[j, pl.ds(l0, 16)]
            for k in range(16):
                pltpu.async_copy(mrow_hbm.at[vl[k]], mail_ref.at[vi[k]], sem)

        @pl.loop(0, 128)
        def _(l):
            pltpu.make_async_copy(mrow_hbm.at[0], mail_ref.at[0], sem).wait()


def _dense_body(pm_ref, m_ref, pts_ref, mts_ref, rts_ref, ef_ref,
                wm_ref, wt_ref, whh_ref, b_ref, tw_ref, tb_ref,
                tlw_ref, tlb_ref, lng_ref, lnb_ref,
                eps_ref, epsb_ref, epd_ref, epdb_ref, epo_ref, epob_ref,
                nrm_ref, mail_ref, pos_ref, neg_ref):
    n = 3 * 512
    pts = pts_ref[...].reshape(n, 1)
    mts = mts_ref[...].reshape(n, 1)
    dt = mts - pts
    tf = jnp.cos(dt * tw_ref[...] + tb_ref[...])
    x = (jnp.dot(m_ref[...].reshape(n, MAIL_DIM), wm_ref[...],
                 preferred_element_type=f32)
         + jnp.dot(tf, wt_ref[...], preferred_element_type=f32)
         + jnp.dot(pm_ref[...].reshape(n, DIM_EMBED), whh_ref[...],
                   preferred_element_type=f32)
         + b_ref[...])
    upd = jnp.tanh(x)
    mu = jnp.mean(upd, axis=-1, keepdims=True)
    var = jnp.mean((upd - mu) ** 2, axis=-1, keepdims=True)
    nrm = (upd - mu) / jnp.sqrt(var + 1e-5) * lng_ref[...] + lnb_ref[...]
    nrm3 = nrm.reshape(3, 512, DIM_EMBED)
    nrm_ref[...] = nrm3

    me = jnp.stack([nrm3[1], nrm3[0]])
    ef = jnp.broadcast_to(ef_ref[...][None], (2, 512, DIM_EDGE))
    mail_ref[...] = jnp.concatenate([me, ef], axis=-1)

    rt = jnp.broadcast_to(rts_ref[...].reshape(1, 512, 1), (3, 512, 1))
    rt = rt.reshape(n, 1)
    td = (rt - mts) / (rt + 1.0)
    proj = nrm * (1.0 + td * tlw_ref[...] + tlb_ref[...])
    p3 = proj.reshape(3, 512, DIM_EMBED)

    a = (jnp.dot(p3[0], eps_ref[...], preferred_element_type=f32)
         + epsb_ref[...])
    hp = jax.nn.relu(a + jnp.dot(p3[1], epd_ref[...],
                                 preferred_element_type=f32) + epdb_ref[...])
    hn = jax.nn.relu(a + jnp.dot(p3[2], epd_ref[...],
                                 preferred_element_type=f32) + epdb_ref[...])
    pos_ref[...] = (jnp.dot(hp, epo_ref[...], preferred_element_type=f32)
                    + epob_ref[...])
    neg_ref[...] = (jnp.dot(hn, epo_ref[...], preferred_element_type=f32)
                    + epob_ref[...])


def _full(shape):
    return pl.BlockSpec(shape, lambda g: tuple(0 for _ in shape))


def _make_tc_dense(interpret=False):
  return pl.pallas_call(
    _dense_body,
    interpret=interpret,
    grid=(B // 512,),
    in_specs=[
        pl.BlockSpec((3, 512, DIM_EMBED), lambda g: (0, g, 0)),
        pl.BlockSpec((3, 512, MAIL_DIM), lambda g: (0, g, 0)),
        pl.BlockSpec((3, 512, 1), lambda g: (0, g, 0)),
        pl.BlockSpec((3, 512, 1), lambda g: (0, g, 0)),
        pl.BlockSpec((512, 1), lambda g: (g, 0)),
        pl.BlockSpec((512, DIM_EDGE), lambda g: (g, 0)),
        _full((MAIL_DIM, DIM_EMBED)),
        _full((DIM_TIME, DIM_EMBED)),
        _full((DIM_EMBED, DIM_EMBED)),
        _full((1, DIM_EMBED)),
        _full((1, DIM_TIME)),
        _full((1, DIM_TIME)),
        _full((1, DIM_EMBED)),
        _full((1, DIM_EMBED)),
        _full((1, DIM_EMBED)),
        _full((1, DIM_EMBED)),
        _full((DIM_EMBED, DIM_EMBED)),
        _full((1, DIM_EMBED)),
        _full((DIM_EMBED, DIM_EMBED)),
        _full((1, DIM_EMBED)),
        _full((DIM_EMBED, 1)),
        _full((1, 1)),
    ],
    out_specs=[
        pl.BlockSpec((3, 512, DIM_EMBED), lambda g: (0, g, 0)),
        pl.BlockSpec((2, 512, MAIL_DIM), lambda g: (0, g, 0)),
        pl.BlockSpec((512, 1), lambda g: (g, 0)),
        pl.BlockSpec((512, 1), lambda g: (g, 0)),
    ],
    out_shape=[
        jax.ShapeDtypeStruct((3, B, DIM_EMBED), f32),
        jax.ShapeDtypeStruct((2, B, MAIL_DIM), f32),
        jax.ShapeDtypeStruct((B, 1), f32),
        jax.ShapeDtypeStruct((B, 1), f32),
    ],
  )


_tc_dense = _make_tc_dense()


def kernel(dst_nodes, root_ts, root_edge_feat, memory, memory_ts, mail,
           mail_ts, W_ih, b_ih, W_hh, b_hh, time_w, time_b, tl_W, tl_b,
           ln_g, ln_b, ep_src_W, ep_src_b, ep_dst_W, ep_dst_b, ep_out_W,
           ep_out_b):
    idx_all = dst_nodes.astype(i32)
    idx_g = idx_all.reshape(NW, GCH, 128)
    idx_s = idx_all[:N2].reshape(NW, SCH, 128)

    pm, m, pts, mtsg = _sc_gather(memory, mail, memory_ts, mail_ts, idx_g)
    lp = _sc_lastpos(idx_all[:N2])

    wihT = W_ih.T
    outs = _tc_dense(
        pm.reshape(3, B, DIM_EMBED), m.reshape(3, B, MAIL_DIM),
        pts.reshape(3, B, 1), mtsg.reshape(3, B, 1),
        root_ts.reshape(B, 1), root_edge_feat,
        wihT[:MAIL_DIM], wihT[MAIL_DIM:], W_hh.T,
        (b_ih + b_hh).reshape(1, DIM_EMBED),
        time_w.reshape(1, DIM_TIME), time_b.reshape(1, DIM_TIME),
        tl_W.reshape(1, DIM_EMBED), tl_b.reshape(1, DIM_EMBED),
        ln_g.reshape(1, DIM_EMBED), ln_b.reshape(1, DIM_EMBED),
        ep_src_W.T, ep_src_b.reshape(1, DIM_EMBED),
        ep_dst_W.T, ep_dst_b.reshape(1, DIM_EMBED),
        ep_out_W.T, ep_out_b.reshape(1, 1),
    )
    nrm3, mail3, pos_scores, neg_scores = outs

    mem_ref = jax.new_ref(memory)
    mail_ref = jax.new_ref(mail)
    mts_ref = jax.new_ref(memory_ts)
    lts_ref = jax.new_ref(mail_ts)
    _sc_scatter(mem_ref, mail_ref, mts_ref, lts_ref,
                nrm3.reshape(N3, DIM_EMBED), mail3.reshape(N2, MAIL_DIM),
                mtsg, jnp.concatenate([root_ts, root_ts]),
                idx_s, lp.reshape(NW, SCH, 128))

    return (pos_scores, neg_scores, mem_ref[...], mts_ref[...],
            mail_ref[...], lts_ref[...])
